# Initial kernel scaffold; baseline (speedup 1.0000x reference)
import jax
import jax.numpy as jnp
from jax.experimental import pallas as pl


def kernel(x, gate_w, gate_b, eW1, eb1, eW2, eb2, sW1, sb1, sW2, sb2):
    raise NotImplementedError("write your pallas kernel here")


# speedup vs baseline: 9.4694x; 9.4694x over previous
import functools

import jax
import jax.numpy as jnp
from jax import lax
from jax.experimental import pallas as pl
from jax.experimental.pallas import tpu as pltpu
from jax.experimental.pallas import tpu_sc as plsc

IN_F = 768
OUT_F = 768
HID = 64
E = 64
B = 4
S = 8192
T = B * S
BLK = 1024
NBLK = T // BLK
TILE_M = 256
NTILES = T // TILE_M + E
PADDED = NTILES * TILE_M
NW = 32
TPW = T // NW
P3C = 128
P5C = 64


def _p1_body(x_ref, gw_ref, gb_ref, w1_ref, b1_ref, w2_ref, b2_ref,
             ew_ref, eid_ref, rank_ref, hist_ref, base_ref):
    xb = x_ref[...]
    logits = jnp.dot(xb, gw_ref[...], preferred_element_type=jnp.float32)
    logits = logits + gb_ref[...]
    m = jnp.max(logits, axis=-1, keepdims=True)
    ex = jnp.exp(logits - m)
    ew_ref[...] = ex / jnp.sum(ex, axis=-1, keepdims=True)
    eid = jnp.argmax(logits, axis=-1).astype(jnp.int32)
    oh = (eid[:, None] == lax.broadcasted_iota(jnp.int32, (1, E), 1))
    oh = oh.astype(jnp.float32)
    r = lax.broadcasted_iota(jnp.int32, (BLK, BLK), 0)
    c = lax.broadcasted_iota(jnp.int32, (BLK, BLK), 1)
    tril = (r > c).astype(jnp.float32)
    cum = jnp.dot(tril, oh, preferred_element_type=jnp.float32)
    rank = jnp.sum(cum * oh, axis=-1).astype(jnp.int32)
    hist = jnp.sum(oh, axis=0).astype(jnp.int32)
    eid_ref[0, 0, :] = eid
    rank_ref[0, 0, :] = rank
    hist_ref[0, 0, :] = hist
    h = jnp.dot(xb, w1_ref[...], preferred_element_type=jnp.float32)
    h = jnp.maximum(h + b1_ref[...], 0.0)
    base_ref[...] = jnp.dot(h, w2_ref[...], preferred_element_type=jnp.float32) + b2_ref[...]


_p1 = pl.pallas_call(
    _p1_body,
    grid=(NBLK,),
    in_specs=[
        pl.BlockSpec((BLK, IN_F), lambda i: (i, 0)),
        pl.BlockSpec((IN_F, E), lambda i: (0, 0)),
        pl.BlockSpec((1, E), lambda i: (0, 0)),
        pl.BlockSpec((IN_F, HID), lambda i: (0, 0)),
        pl.BlockSpec((1, HID), lambda i: (0, 0)),
        pl.BlockSpec((HID, OUT_F), lambda i: (0, 0)),
        pl.BlockSpec((1, OUT_F), lambda i: (0, 0)),
    ],
    out_specs=[
        pl.BlockSpec((BLK, E), lambda i: (i, 0)),
        pl.BlockSpec((1, 1, BLK), lambda i: (i, 0, 0)),
        pl.BlockSpec((1, 1, BLK), lambda i: (i, 0, 0)),
        pl.BlockSpec((1, 1, E), lambda i: (i, 0, 0)),
        pl.BlockSpec((BLK, OUT_F), lambda i: (i, 0)),
    ],
    out_shape=[
        jax.ShapeDtypeStruct((T, E), jnp.float32),
        jax.ShapeDtypeStruct((NBLK, 1, BLK), jnp.int32),
        jax.ShapeDtypeStruct((NBLK, 1, BLK), jnp.int32),
        jax.ShapeDtypeStruct((NBLK, 1, E), jnp.int32),
        jax.ShapeDtypeStruct((T, OUT_F), jnp.float32),
    ],
)


def _p2_body(hist_ref, eid_ref, rank_ref, pos_ref, te_ref):
    b = pl.program_id(0)
    h = hist_ref[...].reshape(NBLK, E).astype(jnp.float32)
    counts = jnp.sum(h, axis=0, keepdims=True)
    padded = jnp.ceil(counts / TILE_M) * TILE_M
    rr = lax.broadcasted_iota(jnp.int32, (E, E), 0)
    cc = lax.broadcasted_iota(jnp.int32, (E, E), 1)
    triu = (rr <= cc).astype(jnp.float32)
    pcum = jnp.dot(padded, triu, preferred_element_type=jnp.float32)
    poff = pcum - padded
    mask = (lax.broadcasted_iota(jnp.int32, (1, NBLK), 1) < b).astype(jnp.float32)
    carry = jnp.dot(mask, h, preferred_element_type=jnp.float32)
    db = poff + carry
    eid = eid_ref[0, 0, :]
    rank = rank_ref[0, 0, :]
    oh = (eid[:, None] == lax.broadcasted_iota(jnp.int32, (1, E), 1))
    pos = jnp.sum(oh.astype(jnp.float32) * db, axis=-1).astype(jnp.int32) + rank
    pos_ref[0, 0, :] = pos
    f = (lax.broadcasted_iota(jnp.int32, (8, 128), 0) * 128
         + lax.broadcasted_iota(jnp.int32, (8, 128), 1))
    p = (f * TILE_M).astype(jnp.float32)
    acc = jnp.zeros((8, 128), jnp.float32)
    for e in range(E):
        pe = lax.slice(poff, (0, e), (1, e + 1))
        acc = acc + (p >= pe).astype(jnp.float32)
    te_ref[...] = (acc - 1.0).astype(jnp.int32)


_p2 = pl.pallas_call(
    _p2_body,
    grid=(NBLK,),
    in_specs=[
        pl.BlockSpec((NBLK, 1, E), lambda i: (0, 0, 0)),
        pl.BlockSpec((1, 1, BLK), lambda i: (i, 0, 0)),
        pl.BlockSpec((1, 1, BLK), lambda i: (i, 0, 0)),
    ],
    out_specs=[
        pl.BlockSpec((1, 1, BLK), lambda i: (i, 0, 0)),
        pl.BlockSpec((8, 128), lambda i: (0, 0)),
    ],
    out_shape=[
        jax.ShapeDtypeStruct((NBLK, 1, BLK), jnp.int32),
        jax.ShapeDtypeStruct((8, 128), jnp.int32),
    ],
)


def _p3_body(x_hbm, pos_hbm, xs_hbm, rows_v, pos_v, sem):
    w = lax.axis_index("s") * 2 + lax.axis_index("c")
    for cch in range(TPW // P3C):
        tok0 = w * TPW + cch * P3C
        pltpu.sync_copy(x_hbm.at[pl.ds(tok0, P3C)], rows_v)
        pltpu.sync_copy(pos_hbm.at[pl.ds(tok0, P3C)], pos_v)
        pltpu.async_copy(rows_v, xs_hbm.at[pos_v], sem).wait()


@functools.cache
def _sc_kernels():
    mesh = plsc.VectorSubcoreMesh(core_axis_name="c", subcore_axis_name="s")
    p3 = pl.kernel(
        _p3_body,
        out_type=jax.ShapeDtypeStruct((PADDED, IN_F), jnp.float32),
        mesh=mesh,
        scratch_types=(pltpu.VMEM((P3C, IN_F), jnp.float32),
                       pltpu.VMEM((P3C,), jnp.int32),
                       pltpu.SemaphoreType.DMA),
    )
    p5 = pl.kernel(
        _p5_body,
        out_type=jax.ShapeDtypeStruct((T, OUT_F), jnp.float32),
        mesh=mesh,
        scratch_types=(pltpu.VMEM((P5C, OUT_F), jnp.float32),
                       pltpu.VMEM((P5C, OUT_F), jnp.float32),
                       pltpu.VMEM((P5C,), jnp.int32),
                       pltpu.SemaphoreType.DMA),
    )
    return p3, p5


def _p4_body(tid_ref, xs_ref, w1_ref, b1_ref, w2_ref, b2_ref, ys_ref):
    xb = xs_ref[...]
    h = jnp.dot(xb, w1_ref[0], preferred_element_type=jnp.float32)
    h = jnp.maximum(h + b1_ref[0], 0.0)
    ys_ref[...] = jnp.dot(h, w2_ref[0], preferred_element_type=jnp.float32) + b2_ref[0]


_p4 = pl.pallas_call(
    _p4_body,
    grid_spec=pltpu.PrefetchScalarGridSpec(
        num_scalar_prefetch=1,
        grid=(NTILES,),
        in_specs=[
            pl.BlockSpec((TILE_M, IN_F), lambda i, tid: (i, 0)),
            pl.BlockSpec((1, IN_F, HID), lambda i, tid: (tid[i], 0, 0)),
            pl.BlockSpec((1, 1, HID), lambda i, tid: (tid[i], 0, 0)),
            pl.BlockSpec((1, HID, OUT_F), lambda i, tid: (tid[i], 0, 0)),
            pl.BlockSpec((1, 1, OUT_F), lambda i, tid: (tid[i], 0, 0)),
        ],
        out_specs=pl.BlockSpec((TILE_M, OUT_F), lambda i, tid: (i, 0)),
    ),
    out_shape=jax.ShapeDtypeStruct((PADDED, OUT_F), jnp.float32),
)


def _p5_body(ys_hbm, base_hbm, pos_hbm, out_hbm, yr_v, br_v, pos_v, sem):
    w = lax.axis_index("s") * 2 + lax.axis_index("c")
    for cch in range(TPW // P5C):
        tok0 = w * TPW + cch * P5C
        pltpu.sync_copy(pos_hbm.at[pl.ds(tok0, P5C)], pos_v)
        cp = pltpu.async_copy(ys_hbm.at[pos_v], yr_v, sem)
        pltpu.sync_copy(base_hbm.at[pl.ds(tok0, P5C)], br_v)
        cp.wait()

        def row_add(i, _):
            for j in range(OUT_F // 16):
                sl = pl.ds(j * 16, 16)
                br_v[i, sl] = br_v[i, sl] + yr_v[i, sl]
            return 0

        lax.fori_loop(0, P5C, row_add, 0)
        pltpu.sync_copy(br_v, out_hbm.at[pl.ds(tok0, P5C)])


def kernel(x, gate_w, gate_b, eW1, eb1, eW2, eb2, sW1, sb1, sW2, sb2):
    Bq, Sq, Dd = x.shape
    xf = x.reshape(T, IN_F)
    ew, eid3, rank3, hist3, base = _p1(
        xf, gate_w, gate_b.reshape(1, E),
        sW1[0], sb1.reshape(1, HID), sW2[0], sb2.reshape(1, OUT_F))
    pos3, te = _p2(hist3, eid3, rank3)
    pos = pos3.reshape(T)
    p3, p5 = _sc_kernels()
    xs = p3(xf, pos)
    tile_eid = te.reshape(1024)[:NTILES]
    ys = _p4(tile_eid, xs, eW1, eb1.reshape(E, 1, HID), eW2,
             eb2.reshape(E, 1, OUT_F))
    out = p5(ys, base, pos)
    return out.reshape(Bq, Sq, OUT_F), ew


# speedup vs baseline: 10.6696x; 1.1267x over previous
import functools

import jax
import jax.numpy as jnp
import numpy as np
from jax import lax
from jax.experimental import pallas as pl
from jax.experimental.pallas import tpu as pltpu
from jax.experimental.pallas import tpu_sc as plsc

IN_F = 768
OUT_F = 768
HID = 64
E = 64
B = 4
S = 8192
T = B * S
BLK = 1024
NBLK = T // BLK
TILE_M = 256
NTILES = T // TILE_M + E
PADDED = NTILES * TILE_M
NW = 32
TPW = T // NW
P3C = 128
P5C = 128


_TRIL = np.tril(np.ones((BLK, BLK), np.float32), -1).astype(jax.numpy.bfloat16)


def _p1_body(x_ref, gw_ref, gb_ref, tril_ref,
             ew_ref, eid_ref, rank_ref, hist_ref):
    xb = x_ref[...]
    logits = jnp.dot(xb, gw_ref[...], preferred_element_type=jnp.float32)
    logits = logits + gb_ref[...]
    m = jnp.max(logits, axis=-1, keepdims=True)
    ex = jnp.exp(logits - m)
    ew_ref[...] = ex / jnp.sum(ex, axis=-1, keepdims=True)
    eid = jnp.argmax(logits, axis=-1).astype(jnp.int32)
    oh = (eid[:, None] == lax.broadcasted_iota(jnp.int32, (1, E), 1))
    ohb = oh.astype(jnp.bfloat16)
    cum = jnp.dot(tril_ref[...], ohb, preferred_element_type=jnp.float32)
    rank = jnp.sum(cum * oh.astype(jnp.float32), axis=-1).astype(jnp.int32)
    hist = jnp.sum(oh.astype(jnp.float32), axis=0).astype(jnp.int32)
    eid_ref[0, 0, :] = eid
    rank_ref[0, 0, :] = rank
    hist_ref[0, 0, :] = hist


_p1 = pl.pallas_call(
    _p1_body,
    grid=(NBLK,),
    in_specs=[
        pl.BlockSpec((BLK, IN_F), lambda i: (i, 0)),
        pl.BlockSpec((IN_F, E), lambda i: (0, 0)),
        pl.BlockSpec((1, E), lambda i: (0, 0)),
        pl.BlockSpec((BLK, BLK), lambda i: (0, 0)),
    ],
    out_specs=[
        pl.BlockSpec((BLK, E), lambda i: (i, 0)),
        pl.BlockSpec((1, 1, BLK), lambda i: (i, 0, 0)),
        pl.BlockSpec((1, 1, BLK), lambda i: (i, 0, 0)),
        pl.BlockSpec((1, 1, E), lambda i: (i, 0, 0)),
    ],
    out_shape=[
        jax.ShapeDtypeStruct((T, E), jnp.float32),
        jax.ShapeDtypeStruct((NBLK, 1, BLK), jnp.int32),
        jax.ShapeDtypeStruct((NBLK, 1, BLK), jnp.int32),
        jax.ShapeDtypeStruct((NBLK, 1, E), jnp.int32),
    ],
)


def _p2_body(hist_ref, eid_ref, rank_ref, pos_ref, te_ref):
    b = pl.program_id(0)
    h = hist_ref[...].reshape(NBLK, E).astype(jnp.float32)
    counts = jnp.sum(h, axis=0, keepdims=True)
    padded = jnp.ceil(counts / TILE_M) * TILE_M
    rr = lax.broadcasted_iota(jnp.int32, (E, E), 0)
    cc = lax.broadcasted_iota(jnp.int32, (E, E), 1)
    triu = (rr <= cc).astype(jnp.float32)
    pcum = jnp.dot(padded, triu, preferred_element_type=jnp.float32)
    poff = pcum - padded
    mask = (lax.broadcasted_iota(jnp.int32, (1, NBLK), 1) < b).astype(jnp.float32)
    carry = jnp.dot(mask, h, preferred_element_type=jnp.float32)
    db = poff + carry
    eid = eid_ref[0, 0, :]
    rank = rank_ref[0, 0, :]
    oh = (eid[:, None] == lax.broadcasted_iota(jnp.int32, (1, E), 1))
    pos = jnp.sum(oh.astype(jnp.float32) * db, axis=-1).astype(jnp.int32) + rank
    pos_ref[0, 0, :] = pos
    @pl.when(b == 0)
    def _():
        f = (lax.broadcasted_iota(jnp.int32, (8, 128), 0) * 128
             + lax.broadcasted_iota(jnp.int32, (8, 128), 1))
        p = (f * TILE_M).astype(jnp.float32)
        acc = jnp.zeros((8, 128), jnp.float32)
        for e in range(E):
            pe = lax.slice(poff, (0, e), (1, e + 1))
            acc = acc + (p >= pe).astype(jnp.float32)
        te_ref[...] = (acc - 1.0).astype(jnp.int32)


_p2 = pl.pallas_call(
    _p2_body,
    grid=(NBLK,),
    in_specs=[
        pl.BlockSpec((NBLK, 1, E), lambda i: (0, 0, 0)),
        pl.BlockSpec((1, 1, BLK), lambda i: (i, 0, 0)),
        pl.BlockSpec((1, 1, BLK), lambda i: (i, 0, 0)),
    ],
    out_specs=[
        pl.BlockSpec((1, 1, BLK), lambda i: (i, 0, 0)),
        pl.BlockSpec((8, 128), lambda i: (0, 0)),
    ],
    out_shape=[
        jax.ShapeDtypeStruct((NBLK, 1, BLK), jnp.int32),
        jax.ShapeDtypeStruct((8, 128), jnp.int32),
    ],
)


def _p3_body(x_hbm, pos_hbm, xs_hbm, rows_v, pos_v, sem):
    w = lax.axis_index("s") * 2 + lax.axis_index("c")
    for cch in range(TPW // P3C):
        tok0 = w * TPW + cch * P3C
        pltpu.sync_copy(x_hbm.at[pl.ds(tok0, P3C)], rows_v)
        pltpu.sync_copy(pos_hbm.at[pl.ds(tok0, P3C)], pos_v)
        pltpu.async_copy(rows_v, xs_hbm.at[pos_v], sem).wait()


@functools.cache
def _sc_kernels():
    mesh = plsc.VectorSubcoreMesh(core_axis_name="c", subcore_axis_name="s")
    p3 = pl.kernel(
        _p3_body,
        out_type=jax.ShapeDtypeStruct((PADDED, IN_F), jnp.float32),
        mesh=mesh,
        scratch_types=(pltpu.VMEM((P3C, IN_F), jnp.float32),
                       pltpu.VMEM((P3C,), jnp.int32),
                       pltpu.SemaphoreType.DMA),
    )
    p5 = pl.kernel(
        _p5_body,
        out_type=jax.ShapeDtypeStruct((T, OUT_F), jnp.float32),
        mesh=mesh,
        scratch_types=(pltpu.VMEM((P5C, OUT_F), jnp.float32),
                       pltpu.VMEM((P5C,), jnp.int32),
                       pltpu.SemaphoreType.DMA),
    )
    return p3, p5


def _p4_body(tid_ref, xs_ref, w1_ref, b1_ref, w2_ref, b2_ref,
             sw1_ref, sb1_ref, sw2_ref, sb2_ref, ys_ref):
    xb = xs_ref[...]
    h = jnp.dot(xb, w1_ref[0], preferred_element_type=jnp.float32)
    h = jnp.maximum(h + b1_ref[0], 0.0)
    y = jnp.dot(h, w2_ref[0], preferred_element_type=jnp.float32) + b2_ref[0]
    hs = jnp.dot(xb, sw1_ref[...], preferred_element_type=jnp.float32)
    hs = jnp.maximum(hs + sb1_ref[...], 0.0)
    ys_ref[...] = y + jnp.dot(hs, sw2_ref[...],
                              preferred_element_type=jnp.float32) + sb2_ref[...]


_p4 = pl.pallas_call(
    _p4_body,
    grid_spec=pltpu.PrefetchScalarGridSpec(
        num_scalar_prefetch=1,
        grid=(NTILES,),
        in_specs=[
            pl.BlockSpec((TILE_M, IN_F), lambda i, tid: (i, 0)),
            pl.BlockSpec((1, IN_F, HID), lambda i, tid: (tid[i], 0, 0)),
            pl.BlockSpec((1, 1, HID), lambda i, tid: (tid[i], 0, 0)),
            pl.BlockSpec((1, HID, OUT_F), lambda i, tid: (tid[i], 0, 0)),
            pl.BlockSpec((1, 1, OUT_F), lambda i, tid: (tid[i], 0, 0)),
            pl.BlockSpec((IN_F, HID), lambda i, tid: (0, 0)),
            pl.BlockSpec((1, HID), lambda i, tid: (0, 0)),
            pl.BlockSpec((HID, OUT_F), lambda i, tid: (0, 0)),
            pl.BlockSpec((1, OUT_F), lambda i, tid: (0, 0)),
        ],
        out_specs=pl.BlockSpec((TILE_M, OUT_F), lambda i, tid: (i, 0)),
    ),
    out_shape=jax.ShapeDtypeStruct((PADDED, OUT_F), jnp.float32),
)


def _p5_body(ys_hbm, pos_hbm, out_hbm, yr_v, pos_v, sem):
    w = lax.axis_index("s") * 2 + lax.axis_index("c")
    for cch in range(TPW // P5C):
        tok0 = w * TPW + cch * P5C
        pltpu.sync_copy(pos_hbm.at[pl.ds(tok0, P5C)], pos_v)
        pltpu.async_copy(ys_hbm.at[pos_v], yr_v, sem).wait()
        pltpu.sync_copy(yr_v, out_hbm.at[pl.ds(tok0, P5C)])


def kernel(x, gate_w, gate_b, eW1, eb1, eW2, eb2, sW1, sb1, sW2, sb2):
    Bq, Sq, Dd = x.shape
    xf = x.reshape(T, IN_F)
    ew, eid3, rank3, hist3 = _p1(
        xf, gate_w, gate_b.reshape(1, E), jnp.asarray(_TRIL))
    pos3, te = _p2(hist3, eid3, rank3)
    pos = pos3.reshape(T)
    p3, p5 = _sc_kernels()
    xs = p3(xf, pos)
    tile_eid = te.reshape(1024)[:NTILES]
    ys = _p4(tile_eid, xs, eW1, eb1.reshape(E, 1, HID), eW2,
             eb2.reshape(E, 1, OUT_F), sW1[0], sb1.reshape(1, HID),
             sW2[0], sb2.reshape(1, OUT_F))
    out = p5(ys, pos)
    return out.reshape(Bq, Sq, OUT_F), ew


# speedup vs baseline: 12.2160x; 1.1449x over previous
import functools

import jax
import jax.numpy as jnp
import numpy as np
from jax import lax
from jax.experimental import pallas as pl
from jax.experimental.pallas import tpu as pltpu
from jax.experimental.pallas import tpu_sc as plsc

IN_F = 768
OUT_F = 768
HID = 64
E = 64
B = 4
S = 8192
T = B * S
BLK = 1024
NBLK = T // BLK
TILE_M = 256
NTILES = T // TILE_M + E
PADDED = NTILES * TILE_M
NW = 32
TPW = T // NW
P3C = 128
P5C = 128


SB = 128
NSB = BLK // SB
_TRIL3 = np.ascontiguousarray(
    np.broadcast_to(np.tril(np.ones((SB, SB), np.float32), -1),
                    (NSB, SB, SB))).astype(jax.numpy.bfloat16)
_TRIL8 = np.tril(np.ones((NSB, NSB), np.float32), -1)


def _p1_body(x_ref, gw_ref, gb_ref, tril3_ref, tril8_ref,
             ew_ref, eid_ref, rank_ref, hist_ref):
    xb = x_ref[...]
    logits = jnp.dot(xb, gw_ref[...], preferred_element_type=jnp.float32)
    logits = logits + gb_ref[...]
    m = jnp.max(logits, axis=-1, keepdims=True)
    ex = jnp.exp(logits - m)
    ew_ref[...] = ex / jnp.sum(ex, axis=-1, keepdims=True)
    eid = jnp.argmax(logits, axis=-1).astype(jnp.int32)
    eidl = eid.reshape(NSB, SB)
    eid_ref[0] = eidl
    oh3 = (eidl[:, :, None] == lax.broadcasted_iota(jnp.int32, (1, 1, E), 2))
    oh3f = oh3.astype(jnp.float32)
    cum3 = lax.dot_general(tril3_ref[...], oh3.astype(jnp.bfloat16),
                           (((2,), (1,)), ((0,), (0,))),
                           preferred_element_type=jnp.float32)
    subrank = jnp.sum(cum3 * oh3f, axis=-1)
    hist_s = jnp.sum(oh3f, axis=1)
    carry = jnp.dot(tril8_ref[...], hist_s,
                    preferred_element_type=jnp.float32)
    carsum = jnp.sum(oh3f * carry[:, None, :], axis=-1)
    rank_ref[0] = (subrank + carsum).astype(jnp.int32)
    hist_ref[0, 0, :] = jnp.sum(hist_s, axis=0).astype(jnp.int32)


_p1 = pl.pallas_call(
    _p1_body,
    grid=(NBLK,),
    in_specs=[
        pl.BlockSpec((BLK, IN_F), lambda i: (i, 0)),
        pl.BlockSpec((IN_F, E), lambda i: (0, 0)),
        pl.BlockSpec((1, E), lambda i: (0, 0)),
        pl.BlockSpec((NSB, SB, SB), lambda i: (0, 0, 0)),
        pl.BlockSpec((NSB, NSB), lambda i: (0, 0)),
    ],
    out_specs=[
        pl.BlockSpec((BLK, E), lambda i: (i, 0)),
        pl.BlockSpec((1, NSB, SB), lambda i: (i, 0, 0)),
        pl.BlockSpec((1, NSB, SB), lambda i: (i, 0, 0)),
        pl.BlockSpec((1, 1, E), lambda i: (i, 0, 0)),
    ],
    out_shape=[
        jax.ShapeDtypeStruct((T, E), jnp.float32),
        jax.ShapeDtypeStruct((NBLK, NSB, SB), jnp.int32),
        jax.ShapeDtypeStruct((NBLK, NSB, SB), jnp.int32),
        jax.ShapeDtypeStruct((NBLK, 1, E), jnp.int32),
    ],
)


def _p2_body(hist_ref, eid_ref, rank_ref, pos_ref, te_ref):
    b = pl.program_id(0)
    h = hist_ref[...].reshape(NBLK, E).astype(jnp.float32)
    counts = jnp.sum(h, axis=0, keepdims=True)
    padded = jnp.ceil(counts / TILE_M) * TILE_M
    rr = lax.broadcasted_iota(jnp.int32, (E, E), 0)
    cc = lax.broadcasted_iota(jnp.int32, (E, E), 1)
    triu = (rr <= cc).astype(jnp.float32)
    pcum = jnp.dot(padded, triu, preferred_element_type=jnp.float32)
    poff = pcum - padded
    mask = (lax.broadcasted_iota(jnp.int32, (1, NBLK), 1) < b).astype(jnp.float32)
    carry = jnp.dot(mask, h, preferred_element_type=jnp.float32)
    db = poff + carry
    eidl = eid_ref[0]
    oh3 = (eidl[:, :, None] == lax.broadcasted_iota(jnp.int32, (1, 1, E), 2))
    dbb = db.reshape(1, 1, E)
    pos = jnp.sum(oh3.astype(jnp.float32) * dbb, axis=-1).astype(jnp.int32)
    pos_ref[0] = pos + rank_ref[0]
    @pl.when(b == 0)
    def _():
        f = (lax.broadcasted_iota(jnp.int32, (8, 128), 0) * 128
             + lax.broadcasted_iota(jnp.int32, (8, 128), 1))
        p = (f * TILE_M).astype(jnp.float32)
        acc = jnp.zeros((8, 128), jnp.float32)
        for e in range(E):
            pe = lax.slice(poff, (0, e), (1, e + 1))
            acc = acc + (p >= pe).astype(jnp.float32)
        te_ref[...] = (acc - 1.0).astype(jnp.int32)


_p2 = pl.pallas_call(
    _p2_body,
    grid=(NBLK,),
    in_specs=[
        pl.BlockSpec((NBLK, 1, E), lambda i: (0, 0, 0)),
        pl.BlockSpec((1, NSB, SB), lambda i: (i, 0, 0)),
        pl.BlockSpec((1, NSB, SB), lambda i: (i, 0, 0)),
    ],
    out_specs=[
        pl.BlockSpec((1, NSB, SB), lambda i: (i, 0, 0)),
        pl.BlockSpec((8, 128), lambda i: (0, 0)),
    ],
    out_shape=[
        jax.ShapeDtypeStruct((NBLK, NSB, SB), jnp.int32),
        jax.ShapeDtypeStruct((8, 128), jnp.int32),
    ],
)


def _p3_body(x_hbm, pos_hbm, xs_hbm, rows_v, pos_v, sem):
    w = lax.axis_index("s") * 2 + lax.axis_index("c")
    for cch in range(TPW // P3C):
        tok0 = w * TPW + cch * P3C
        pltpu.sync_copy(x_hbm.at[pl.ds(tok0, P3C)], rows_v)
        pltpu.sync_copy(pos_hbm.at[pl.ds(tok0, P3C)], pos_v)
        pltpu.async_copy(rows_v, xs_hbm.at[pos_v], sem).wait()


@functools.cache
def _sc_kernels():
    mesh = plsc.VectorSubcoreMesh(core_axis_name="c", subcore_axis_name="s")
    p3 = pl.kernel(
        _p3_body,
        out_type=jax.ShapeDtypeStruct((PADDED, IN_F), jnp.float32),
        mesh=mesh,
        scratch_types=(pltpu.VMEM((P3C, IN_F), jnp.float32),
                       pltpu.VMEM((P3C,), jnp.int32),
                       pltpu.SemaphoreType.DMA),
    )
    p5 = pl.kernel(
        _p5_body,
        out_type=jax.ShapeDtypeStruct((T, OUT_F), jnp.float32),
        mesh=mesh,
        scratch_types=(pltpu.VMEM((P5C, OUT_F), jnp.float32),
                       pltpu.VMEM((P5C,), jnp.int32),
                       pltpu.SemaphoreType.DMA),
    )
    return p3, p5


def _p4_body(tid_ref, xs_ref, w1_ref, b1_ref, w2_ref, b2_ref, ys_ref):
    xb = xs_ref[...]
    h = jnp.dot(xb, w1_ref[0], preferred_element_type=jnp.float32)
    h = jnp.maximum(h + b1_ref[0], 0.0)
    ys_ref[...] = jnp.dot(h, w2_ref[0], preferred_element_type=jnp.float32) + b2_ref[0]


_p4 = pl.pallas_call(
    _p4_body,
    grid_spec=pltpu.PrefetchScalarGridSpec(
        num_scalar_prefetch=1,
        grid=(NTILES,),
        in_specs=[
            pl.BlockSpec((TILE_M, IN_F), lambda i, tid: (i, 0)),
            pl.BlockSpec((1, IN_F, 2 * HID), lambda i, tid: (tid[i], 0, 0)),
            pl.BlockSpec((1, 1, 2 * HID), lambda i, tid: (tid[i], 0, 0)),
            pl.BlockSpec((1, 2 * HID, OUT_F), lambda i, tid: (tid[i], 0, 0)),
            pl.BlockSpec((1, 1, OUT_F), lambda i, tid: (tid[i], 0, 0)),
        ],
        out_specs=pl.BlockSpec((TILE_M, OUT_F), lambda i, tid: (i, 0)),
    ),
    out_shape=jax.ShapeDtypeStruct((PADDED, OUT_F), jnp.float32),
)


def _p5_body(ys_hbm, pos_hbm, out_hbm, yr_v, pos_v, sem):
    w = lax.axis_index("s") * 2 + lax.axis_index("c")
    for cch in range(TPW // P5C):
        tok0 = w * TPW + cch * P5C
        pltpu.sync_copy(pos_hbm.at[pl.ds(tok0, P5C)], pos_v)
        pltpu.async_copy(ys_hbm.at[pos_v], yr_v, sem).wait()
        pltpu.sync_copy(yr_v, out_hbm.at[pl.ds(tok0, P5C)])


def kernel(x, gate_w, gate_b, eW1, eb1, eW2, eb2, sW1, sb1, sW2, sb2):
    Bq, Sq, Dd = x.shape
    xf = x.reshape(T, IN_F)
    ew, eid3, rank3, hist3 = _p1(
        xf, gate_w, gate_b.reshape(1, E), jnp.asarray(_TRIL3),
        jnp.asarray(_TRIL8))
    pos3, te = _p2(hist3, eid3, rank3)
    pos = pos3.reshape(T)
    p3, p5 = _sc_kernels()
    xs = p3(xf, pos)
    tile_eid = te.reshape(1024)[:NTILES]
    w1c = jnp.concatenate(
        [eW1, jnp.broadcast_to(sW1[0][None], (E, IN_F, HID))], axis=2)
    b1c = jnp.concatenate(
        [eb1, jnp.broadcast_to(sb1[0][None], (E, HID))], axis=1)
    w2c = jnp.concatenate(
        [eW2, jnp.broadcast_to(sW2[0][None], (E, HID, OUT_F))], axis=1)
    b2c = eb2 + sb2[0][None]
    ys = _p4(tile_eid, xs, w1c, b1c.reshape(E, 1, 2 * HID), w2c,
             b2c.reshape(E, 1, OUT_F))
    out = p5(ys, pos)
    return out.reshape(Bq, Sq, OUT_F), ew


# speedup vs baseline: 12.3159x; 1.0082x over previous
import functools

import jax
import jax.numpy as jnp
import numpy as np
from jax import lax
from jax.experimental import pallas as pl
from jax.experimental.pallas import tpu as pltpu
from jax.experimental.pallas import tpu_sc as plsc

IN_F = 768
OUT_F = 768
HID = 64
E = 64
B = 4
S = 8192
T = B * S
BLK = 1024
NBLK = T // BLK
TILE_M = 256
NTILES = T // TILE_M + E
PADDED = NTILES * TILE_M
NW = 32
TPW = T // NW
P3C = 64
P5C = 64


SB = 128
NSB = BLK // SB
_TRIL3 = np.ascontiguousarray(
    np.broadcast_to(np.tril(np.ones((SB, SB), np.float32), -1),
                    (NSB, SB, SB))).astype(jax.numpy.bfloat16)
_TRIL8 = np.tril(np.ones((NSB, NSB), np.float32), -1)


def _p1_body(x_ref, gw_ref, gb_ref, tril3_ref, tril8_ref,
             ew_ref, eid_ref, rank_ref, hist_ref):
    xb = x_ref[...]
    logits = jnp.dot(xb, gw_ref[...], preferred_element_type=jnp.float32)
    logits = logits + gb_ref[...]
    m = jnp.max(logits, axis=-1, keepdims=True)
    ex = jnp.exp(logits - m)
    ew_ref[...] = ex / jnp.sum(ex, axis=-1, keepdims=True)
    eid = jnp.argmax(logits, axis=-1).astype(jnp.int32)
    eidl = eid.reshape(NSB, SB)
    eid_ref[0] = eidl
    oh3 = (eidl[:, :, None] == lax.broadcasted_iota(jnp.int32, (1, 1, E), 2))
    oh3f = oh3.astype(jnp.float32)
    cum3 = lax.dot_general(tril3_ref[...], oh3.astype(jnp.bfloat16),
                           (((2,), (1,)), ((0,), (0,))),
                           preferred_element_type=jnp.float32)
    subrank = jnp.sum(cum3 * oh3f, axis=-1)
    hist_s = jnp.sum(oh3f, axis=1)
    carry = jnp.dot(tril8_ref[...], hist_s,
                    preferred_element_type=jnp.float32)
    carsum = jnp.sum(oh3f * carry[:, None, :], axis=-1)
    rank_ref[0] = (subrank + carsum).astype(jnp.int32)
    hist_ref[0, 0, :] = jnp.sum(hist_s, axis=0).astype(jnp.int32)


_p1 = pl.pallas_call(
    _p1_body,
    grid=(NBLK,),
    in_specs=[
        pl.BlockSpec((BLK, IN_F), lambda i: (i, 0)),
        pl.BlockSpec((IN_F, E), lambda i: (0, 0)),
        pl.BlockSpec((1, E), lambda i: (0, 0)),
        pl.BlockSpec((NSB, SB, SB), lambda i: (0, 0, 0)),
        pl.BlockSpec((NSB, NSB), lambda i: (0, 0)),
    ],
    out_specs=[
        pl.BlockSpec((BLK, E), lambda i: (i, 0)),
        pl.BlockSpec((1, NSB, SB), lambda i: (i, 0, 0)),
        pl.BlockSpec((1, NSB, SB), lambda i: (i, 0, 0)),
        pl.BlockSpec((1, 1, E), lambda i: (i, 0, 0)),
    ],
    out_shape=[
        jax.ShapeDtypeStruct((T, E), jnp.float32),
        jax.ShapeDtypeStruct((NBLK, NSB, SB), jnp.int32),
        jax.ShapeDtypeStruct((NBLK, NSB, SB), jnp.int32),
        jax.ShapeDtypeStruct((NBLK, 1, E), jnp.int32),
    ],
)


def _p2_body(hist_ref, eid_ref, rank_ref, pos_ref, te_ref):
    b = pl.program_id(0)
    h = hist_ref[...].reshape(NBLK, E).astype(jnp.float32)
    counts = jnp.sum(h, axis=0, keepdims=True)
    padded = jnp.ceil(counts / TILE_M) * TILE_M
    rr = lax.broadcasted_iota(jnp.int32, (E, E), 0)
    cc = lax.broadcasted_iota(jnp.int32, (E, E), 1)
    triu = (rr <= cc).astype(jnp.float32)
    pcum = jnp.dot(padded, triu, preferred_element_type=jnp.float32)
    poff = pcum - padded
    mask = (lax.broadcasted_iota(jnp.int32, (1, NBLK), 1) < b).astype(jnp.float32)
    carry = jnp.dot(mask, h, preferred_element_type=jnp.float32)
    db = poff + carry
    eidl = eid_ref[0]
    oh3 = (eidl[:, :, None] == lax.broadcasted_iota(jnp.int32, (1, 1, E), 2))
    dbb = db.reshape(1, 1, E)
    pos = jnp.sum(oh3.astype(jnp.float32) * dbb, axis=-1).astype(jnp.int32)
    pos_ref[0] = pos + rank_ref[0]
    @pl.when(b == 0)
    def _():
        f = (lax.broadcasted_iota(jnp.int32, (8, 128), 0) * 128
             + lax.broadcasted_iota(jnp.int32, (8, 128), 1))
        p = (f * TILE_M).astype(jnp.float32)
        acc = jnp.zeros((8, 128), jnp.float32)
        for e in range(E):
            pe = lax.slice(poff, (0, e), (1, e + 1))
            acc = acc + (p >= pe).astype(jnp.float32)
        te_ref[...] = (acc - 1.0).astype(jnp.int32)


_p2 = pl.pallas_call(
    _p2_body,
    grid=(NBLK,),
    in_specs=[
        pl.BlockSpec((NBLK, 1, E), lambda i: (0, 0, 0)),
        pl.BlockSpec((1, NSB, SB), lambda i: (i, 0, 0)),
        pl.BlockSpec((1, NSB, SB), lambda i: (i, 0, 0)),
    ],
    out_specs=[
        pl.BlockSpec((1, NSB, SB), lambda i: (i, 0, 0)),
        pl.BlockSpec((8, 128), lambda i: (0, 0)),
    ],
    out_shape=[
        jax.ShapeDtypeStruct((NBLK, NSB, SB), jnp.int32),
        jax.ShapeDtypeStruct((8, 128), jnp.int32),
    ],
)


def _p3_body(x_hbm, pos_hbm, xs_hbm, rows_v, pos_v,
             l0, l1, q0, q1, s0, s1):
    w = lax.axis_index("s") * 2 + lax.axis_index("c")
    base = w * TPW
    lsem, qsem, ssem = [l0, l1], [q0, q1], [s0, s1]
    nch = TPW // P3C

    def load(c):
        buf = c % 2
        lx = pltpu.async_copy(x_hbm.at[pl.ds(base + c * P3C, P3C)],
                              rows_v.at[buf], lsem[buf])
        lp = pltpu.async_copy(pos_hbm.at[pl.ds(base + c * P3C, P3C)],
                              pos_v.at[buf], qsem[buf])
        return lx, lp

    loads = {0: load(0)}
    scats = {}
    for c in range(nch):
        buf = c % 2
        lx, lp = loads.pop(c)
        lx.wait()
        lp.wait()
        if c + 1 < nch:
            if c - 1 >= 0:
                scats.pop(c - 1).wait()
            loads[c + 1] = load(c + 1)
        scats[c] = pltpu.async_copy(rows_v.at[buf],
                                    xs_hbm.at[pos_v.at[buf]], ssem[buf])
    for c in sorted(scats):
        scats.pop(c).wait()


@functools.cache
def _sc_kernels():
    mesh = plsc.VectorSubcoreMesh(core_axis_name="c", subcore_axis_name="s")
    p3 = pl.kernel(
        _p3_body,
        out_type=jax.ShapeDtypeStruct((PADDED, IN_F), jnp.float32),
        mesh=mesh,
        scratch_types=(pltpu.VMEM((2, P3C, IN_F), jnp.float32),
                       pltpu.VMEM((2, P3C), jnp.int32),
                       pltpu.SemaphoreType.DMA, pltpu.SemaphoreType.DMA,
                       pltpu.SemaphoreType.DMA, pltpu.SemaphoreType.DMA,
                       pltpu.SemaphoreType.DMA, pltpu.SemaphoreType.DMA),
    )
    p5 = pl.kernel(
        _p5_body,
        out_type=jax.ShapeDtypeStruct((T, OUT_F), jnp.float32),
        mesh=mesh,
        scratch_types=(pltpu.VMEM((2, P5C, OUT_F), jnp.float32),
                       pltpu.VMEM((2, P5C), jnp.int32),
                       pltpu.SemaphoreType.DMA, pltpu.SemaphoreType.DMA,
                       pltpu.SemaphoreType.DMA, pltpu.SemaphoreType.DMA),
    )
    return p3, p5


def _p4_body(tid_ref, xs_ref, w1_ref, b1_ref, w2_ref, b2_ref, ys_ref):
    xb = xs_ref[...]
    h = jnp.dot(xb, w1_ref[0], preferred_element_type=jnp.float32)
    h = jnp.maximum(h + b1_ref[0], 0.0)
    ys_ref[...] = jnp.dot(h, w2_ref[0], preferred_element_type=jnp.float32) + b2_ref[0]


_p4 = pl.pallas_call(
    _p4_body,
    grid_spec=pltpu.PrefetchScalarGridSpec(
        num_scalar_prefetch=1,
        grid=(NTILES,),
        in_specs=[
            pl.BlockSpec((TILE_M, IN_F), lambda i, tid: (i, 0)),
            pl.BlockSpec((1, IN_F, 2 * HID), lambda i, tid: (tid[i], 0, 0)),
            pl.BlockSpec((1, 1, 2 * HID), lambda i, tid: (tid[i], 0, 0)),
            pl.BlockSpec((1, 2 * HID, OUT_F), lambda i, tid: (tid[i], 0, 0)),
            pl.BlockSpec((1, 1, OUT_F), lambda i, tid: (tid[i], 0, 0)),
        ],
        out_specs=pl.BlockSpec((TILE_M, OUT_F), lambda i, tid: (i, 0)),
    ),
    out_shape=jax.ShapeDtypeStruct((PADDED, OUT_F), jnp.float32),
)


def _p5_body(ys_hbm, pos_hbm, out_hbm, yr_v, pos_v,
             g0, g1, t0, t1):
    w = lax.axis_index("s") * 2 + lax.axis_index("c")
    base = w * TPW
    gsem, tsem = [g0, g1], [t0, t1]
    nch = TPW // P5C

    pltpu.sync_copy(pos_hbm.at[pl.ds(base, P5C)], pos_v.at[0])
    gaths = {0: pltpu.async_copy(ys_hbm.at[pos_v.at[0]], yr_v.at[0], gsem[0])}
    stores = {}
    for c in range(nch):
        buf = c % 2
        gaths.pop(c).wait()
        stores[c] = pltpu.async_copy(
            yr_v.at[buf], out_hbm.at[pl.ds(base + c * P5C, P5C)], tsem[buf])
        if c + 1 < nch:
            ob = 1 - buf
            if c - 1 >= 0:
                stores.pop(c - 1).wait()
            pltpu.sync_copy(pos_hbm.at[pl.ds(base + (c + 1) * P5C, P5C)],
                            pos_v.at[ob])
            gaths[c + 1] = pltpu.async_copy(ys_hbm.at[pos_v.at[ob]],
                                            yr_v.at[ob], gsem[ob])
    for c in sorted(stores):
        stores.pop(c).wait()


def kernel(x, gate_w, gate_b, eW1, eb1, eW2, eb2, sW1, sb1, sW2, sb2):
    Bq, Sq, Dd = x.shape
    xf = x.reshape(T, IN_F)
    ew, eid3, rank3, hist3 = _p1(
        xf, gate_w, gate_b.reshape(1, E), jnp.asarray(_TRIL3),
        jnp.asarray(_TRIL8))
    pos3, te = _p2(hist3, eid3, rank3)
    pos = pos3.reshape(T)
    p3, p5 = _sc_kernels()
    xs = p3(xf, pos)
    tile_eid = te.reshape(1024)[:NTILES]
    w1c = jnp.concatenate(
        [eW1, jnp.broadcast_to(sW1[0][None], (E, IN_F, HID))], axis=2)
    b1c = jnp.concatenate(
        [eb1, jnp.broadcast_to(sb1[0][None], (E, HID))], axis=1)
    w2c = jnp.concatenate(
        [eW2, jnp.broadcast_to(sW2[0][None], (E, HID, OUT_F))], axis=1)
    b2c = eb2 + sb2[0][None]
    ys = _p4(tile_eid, xs, w1c, b1c.reshape(E, 1, 2 * HID), w2c,
             b2c.reshape(E, 1, OUT_F))
    out = p5(ys, pos)
    return out.reshape(Bq, Sq, OUT_F), ew


# speedup vs baseline: 13.0000x; 1.0556x over previous
import functools

import jax
import jax.numpy as jnp
import numpy as np
from jax import lax
from jax.experimental import pallas as pl
from jax.experimental.pallas import tpu as pltpu
from jax.experimental.pallas import tpu_sc as plsc

IN_F = 768
OUT_F = 768
HID = 64
E = 64
B = 4
S = 8192
T = B * S
BLK = 1024
NBLK = T // BLK
TILE_M = 256
NTILES = T // TILE_M + E
PADDED = NTILES * TILE_M
NW = 32
TPW = T // NW
P3C = 64
P5C = 64


SB = 128
NSB = BLK // SB
_TRIL3 = np.ascontiguousarray(
    np.broadcast_to(np.tril(np.ones((SB, SB), np.float32), -1),
                    (NSB, SB, SB))).astype(jax.numpy.bfloat16)
_TRIL8 = np.tril(np.ones((NSB, NSB), np.float32), -1)


def _p1_body(x_ref, gw_ref, gb_ref, tril3_ref, tril8_ref,
             ew_ref, eid_ref, rank_ref, hist_ref):
    xb = x_ref[...]
    logits = jnp.dot(xb, gw_ref[...], preferred_element_type=jnp.float32)
    logits = logits + gb_ref[...]
    m = jnp.max(logits, axis=-1, keepdims=True)
    ex = jnp.exp(logits - m)
    ew_ref[...] = ex / jnp.sum(ex, axis=-1, keepdims=True)
    eid = jnp.argmax(logits, axis=-1).astype(jnp.int32)
    eidl = eid.reshape(NSB, SB)
    eid_ref[0] = eidl
    oh3 = (eidl[:, :, None] == lax.broadcasted_iota(jnp.int32, (1, 1, E), 2))
    oh3f = oh3.astype(jnp.float32)
    cum3 = lax.dot_general(tril3_ref[...], oh3.astype(jnp.bfloat16),
                           (((2,), (1,)), ((0,), (0,))),
                           preferred_element_type=jnp.float32)
    subrank = jnp.sum(cum3 * oh3f, axis=-1)
    hist_s = jnp.sum(oh3f, axis=1)
    carry = jnp.dot(tril8_ref[...], hist_s,
                    preferred_element_type=jnp.float32)
    carsum = jnp.sum(oh3f * carry[:, None, :], axis=-1)
    rank_ref[0] = (subrank + carsum).astype(jnp.int32)
    hist_ref[0, 0, :] = jnp.sum(hist_s, axis=0).astype(jnp.int32)


_p1 = pl.pallas_call(
    _p1_body,
    grid=(NBLK,),
    in_specs=[
        pl.BlockSpec((BLK, IN_F), lambda i: (i, 0)),
        pl.BlockSpec((IN_F, E), lambda i: (0, 0)),
        pl.BlockSpec((1, E), lambda i: (0, 0)),
        pl.BlockSpec((NSB, SB, SB), lambda i: (0, 0, 0)),
        pl.BlockSpec((NSB, NSB), lambda i: (0, 0)),
    ],
    out_specs=[
        pl.BlockSpec((BLK, E), lambda i: (i, 0)),
        pl.BlockSpec((1, NSB, SB), lambda i: (i, 0, 0)),
        pl.BlockSpec((1, NSB, SB), lambda i: (i, 0, 0)),
        pl.BlockSpec((1, 1, E), lambda i: (i, 0, 0)),
    ],
    out_shape=[
        jax.ShapeDtypeStruct((T, E), jnp.float32),
        jax.ShapeDtypeStruct((NBLK, NSB, SB), jnp.int32),
        jax.ShapeDtypeStruct((NBLK, NSB, SB), jnp.int32),
        jax.ShapeDtypeStruct((NBLK, 1, E), jnp.int32),
    ],
)


def _p2_body(hist_ref, eid_ref, rank_ref, pos_ref, te_ref):
    b = pl.program_id(0)
    h = hist_ref[...].reshape(NBLK, E).astype(jnp.float32)
    counts = jnp.sum(h, axis=0, keepdims=True)
    padded = jnp.ceil(counts / TILE_M) * TILE_M
    rr = lax.broadcasted_iota(jnp.int32, (E, E), 0)
    cc = lax.broadcasted_iota(jnp.int32, (E, E), 1)
    triu = (rr <= cc).astype(jnp.float32)
    pcum = jnp.dot(padded, triu, preferred_element_type=jnp.float32)
    poff = pcum - padded
    mask = (lax.broadcasted_iota(jnp.int32, (1, NBLK), 1) < b).astype(jnp.float32)
    carry = jnp.dot(mask, h, preferred_element_type=jnp.float32)
    db = poff + carry
    eidl = eid_ref[0]
    oh3 = (eidl[:, :, None] == lax.broadcasted_iota(jnp.int32, (1, 1, E), 2))
    dbb = db.reshape(1, 1, E)
    pos = jnp.sum(oh3.astype(jnp.float32) * dbb, axis=-1).astype(jnp.int32)
    pos_ref[0] = pos + rank_ref[0]
    @pl.when(b == 0)
    def _():
        f = (lax.broadcasted_iota(jnp.int32, (8, 128), 0) * 128
             + lax.broadcasted_iota(jnp.int32, (8, 128), 1))
        p = (f * TILE_M).astype(jnp.float32)
        acc = jnp.zeros((8, 128), jnp.float32)
        for e in range(E):
            pe = lax.slice(poff, (0, e), (1, e + 1))
            acc = acc + (p >= pe).astype(jnp.float32)
        te_ref[...] = (acc - 1.0).astype(jnp.int32)


_p2 = pl.pallas_call(
    _p2_body,
    grid=(NBLK,),
    in_specs=[
        pl.BlockSpec((NBLK, 1, E), lambda i: (0, 0, 0)),
        pl.BlockSpec((1, NSB, SB), lambda i: (i, 0, 0)),
        pl.BlockSpec((1, NSB, SB), lambda i: (i, 0, 0)),
    ],
    out_specs=[
        pl.BlockSpec((1, NSB, SB), lambda i: (i, 0, 0)),
        pl.BlockSpec((8, 128), lambda i: (0, 0)),
    ],
    out_shape=[
        jax.ShapeDtypeStruct((NBLK, NSB, SB), jnp.int32),
        jax.ShapeDtypeStruct((8, 128), jnp.int32),
    ],
)


def _p3_body(x_hbm, pos_hbm, xs_hbm, rows_v, pos_v,
             l0, l1, q0, q1, s0, s1):
    w = lax.axis_index("s") * 2 + lax.axis_index("c")
    base = w * TPW
    lsem, qsem, ssem = [l0, l1], [q0, q1], [s0, s1]
    nch = TPW // P3C

    def load(c):
        buf = c % 2
        lx = pltpu.async_copy(x_hbm.at[pl.ds(base + c * P3C, P3C)],
                              rows_v.at[buf], lsem[buf])
        lp = pltpu.async_copy(pos_hbm.at[pl.ds(base + c * P3C, P3C)],
                              pos_v.at[buf], qsem[buf])
        return lx, lp

    loads = {0: load(0)}
    scats = {}
    for c in range(nch):
        buf = c % 2
        lx, lp = loads.pop(c)
        lx.wait()
        lp.wait()
        if c + 1 < nch:
            if c - 1 >= 0:
                scats.pop(c - 1).wait()
            loads[c + 1] = load(c + 1)
        scats[c] = pltpu.async_copy(rows_v.at[buf],
                                    xs_hbm.at[pos_v.at[buf]], ssem[buf])
    for c in sorted(scats):
        scats.pop(c).wait()


@functools.cache
def _sc_kernels():
    mesh = plsc.VectorSubcoreMesh(core_axis_name="c", subcore_axis_name="s")
    p3 = pl.kernel(
        _p3_body,
        out_type=jax.ShapeDtypeStruct((PADDED, IN_F), jnp.float32),
        mesh=mesh,
        scratch_types=(pltpu.VMEM((2, P3C, IN_F), jnp.float32),
                       pltpu.VMEM((2, P3C), jnp.int32),
                       pltpu.SemaphoreType.DMA, pltpu.SemaphoreType.DMA,
                       pltpu.SemaphoreType.DMA, pltpu.SemaphoreType.DMA,
                       pltpu.SemaphoreType.DMA, pltpu.SemaphoreType.DMA),
    )
    p5 = pl.kernel(
        _p5_body,
        out_type=jax.ShapeDtypeStruct((T, OUT_F), jnp.float32),
        mesh=mesh,
        scratch_types=(pltpu.VMEM((2, P5C, OUT_F), jnp.float32),
                       pltpu.VMEM((2, P5C), jnp.int32),
                       pltpu.SemaphoreType.DMA, pltpu.SemaphoreType.DMA,
                       pltpu.SemaphoreType.DMA, pltpu.SemaphoreType.DMA),
    )
    return p3, p5


def _p4_body(tid_ref, xs_ref, w1_ref, b1_ref, w2_ref, b2_ref, ys_ref):
    xb = xs_ref[...].astype(jnp.bfloat16)
    h = jnp.dot(xb, w1_ref[0], preferred_element_type=jnp.float32)
    h = jnp.maximum(h + b1_ref[0], 0.0)
    y = jnp.dot(h.astype(jnp.bfloat16), w2_ref[0],
                preferred_element_type=jnp.float32)
    ys_ref[...] = y + b2_ref[0]


def _tid_map(i, tid):
    return (tid[i // 128, i % 128], 0, 0)


_p4 = pl.pallas_call(
    _p4_body,
    grid_spec=pltpu.PrefetchScalarGridSpec(
        num_scalar_prefetch=1,
        grid=(NTILES,),
        in_specs=[
            pl.BlockSpec((TILE_M, IN_F), lambda i, tid: (i, 0)),
            pl.BlockSpec((1, IN_F, 2 * HID), _tid_map),
            pl.BlockSpec((1, 1, 2 * HID), _tid_map),
            pl.BlockSpec((1, 2 * HID, OUT_F), _tid_map),
            pl.BlockSpec((1, 1, OUT_F), _tid_map),
        ],
        out_specs=pl.BlockSpec((TILE_M, OUT_F), lambda i, tid: (i, 0)),
    ),
    out_shape=jax.ShapeDtypeStruct((PADDED, OUT_F), jnp.float32),
)


def _p5_body(ys_hbm, pos_hbm, out_hbm, yr_v, pos_v,
             g0, g1, t0, t1):
    w = lax.axis_index("s") * 2 + lax.axis_index("c")
    base = w * TPW
    gsem, tsem = [g0, g1], [t0, t1]
    nch = TPW // P5C

    pltpu.sync_copy(pos_hbm.at[pl.ds(base, P5C)], pos_v.at[0])
    gaths = {0: pltpu.async_copy(ys_hbm.at[pos_v.at[0]], yr_v.at[0], gsem[0])}
    stores = {}
    for c in range(nch):
        buf = c % 2
        gaths.pop(c).wait()
        stores[c] = pltpu.async_copy(
            yr_v.at[buf], out_hbm.at[pl.ds(base + c * P5C, P5C)], tsem[buf])
        if c + 1 < nch:
            ob = 1 - buf
            if c - 1 >= 0:
                stores.pop(c - 1).wait()
            pltpu.sync_copy(pos_hbm.at[pl.ds(base + (c + 1) * P5C, P5C)],
                            pos_v.at[ob])
            gaths[c + 1] = pltpu.async_copy(ys_hbm.at[pos_v.at[ob]],
                                            yr_v.at[ob], gsem[ob])
    for c in sorted(stores):
        stores.pop(c).wait()


def kernel(x, gate_w, gate_b, eW1, eb1, eW2, eb2, sW1, sb1, sW2, sb2):
    Bq, Sq, Dd = x.shape
    xf = x.reshape(T, IN_F)
    ew, eid3, rank3, hist3 = _p1(
        xf, gate_w, gate_b.reshape(1, E), jnp.asarray(_TRIL3),
        jnp.asarray(_TRIL8))
    pos3, te = _p2(hist3, eid3, rank3)
    pos = pos3.reshape(T)
    p3, p5 = _sc_kernels()
    xs = p3(xf, pos)
    w1c = jnp.concatenate(
        [eW1, jnp.broadcast_to(sW1[0][None], (E, IN_F, HID))],
        axis=2).astype(jnp.bfloat16)
    b1c = jnp.concatenate(
        [eb1, jnp.broadcast_to(sb1[0][None], (E, HID))], axis=1)
    w2c = jnp.concatenate(
        [eW2, jnp.broadcast_to(sW2[0][None], (E, HID, OUT_F))],
        axis=1).astype(jnp.bfloat16)
    b2c = eb2 + sb2[0][None]
    ys = _p4(te, xs, w1c, b1c.reshape(E, 1, 2 * HID), w2c,
             b2c.reshape(E, 1, OUT_F))
    out = p5(ys, pos)
    return out.reshape(Bq, Sq, OUT_F), ew


# speedup vs baseline: 14.0437x; 1.0803x over previous
import functools

import jax
import jax.numpy as jnp
import numpy as np
from jax import lax
from jax.experimental import pallas as pl
from jax.experimental.pallas import tpu as pltpu
from jax.experimental.pallas import tpu_sc as plsc

IN_F = 768
OUT_F = 768
HID = 64
E = 64
B = 4
S = 8192
T = B * S
BLK = 1024
NBLK = T // BLK
TILE_M = 256
NTILES = T // TILE_M + E
PADDED = NTILES * TILE_M
NW = 32
TPW = T // NW
P3C = 64
P5C = 64


SB = 128
NSB = BLK // SB
_TRIL3 = np.ascontiguousarray(
    np.broadcast_to(np.tril(np.ones((SB, SB), np.float32), -1),
                    (NSB, SB, SB))).astype(jax.numpy.bfloat16)
_TRIL8 = np.tril(np.ones((NSB, NSB), np.float32), -1)


def _p1_body(x_ref, gw_ref, gb_ref, tril3_ref, tril8_ref,
             ew_ref, eid_ref, rank_ref, hist_ref, xp_ref):
    xb = x_ref[...]
    ae = lax.bitcast_convert_type(xb[:, :IN_F // 2], jnp.int32)
    ao = lax.bitcast_convert_type(xb[:, IN_F // 2:], jnp.int32)
    re = ae + 0x7FFF + ((ae >> 16) & 1)
    ro = ao + 0x7FFF + ((ao >> 16) & 1)
    word = (re & jnp.int32(0xFFFF0000 - 0x100000000)) | ((ro >> 16) & 0xFFFF)
    xp_ref[...] = lax.bitcast_convert_type(word, jnp.float32)
    logits = jnp.dot(xb, gw_ref[...], preferred_element_type=jnp.float32)
    logits = logits + gb_ref[...]
    m = jnp.max(logits, axis=-1, keepdims=True)
    ex = jnp.exp(logits - m)
    ew_ref[...] = ex / jnp.sum(ex, axis=-1, keepdims=True)
    eid = jnp.argmax(logits, axis=-1).astype(jnp.int32)
    eidl = eid.reshape(NSB, SB)
    eid_ref[0] = eidl
    oh3 = (eidl[:, :, None] == lax.broadcasted_iota(jnp.int32, (1, 1, E), 2))
    oh3f = oh3.astype(jnp.float32)
    cum3 = lax.dot_general(tril3_ref[...], oh3.astype(jnp.bfloat16),
                           (((2,), (1,)), ((0,), (0,))),
                           preferred_element_type=jnp.float32)
    subrank = jnp.sum(cum3 * oh3f, axis=-1)
    hist_s = jnp.sum(oh3f, axis=1)
    carry = jnp.dot(tril8_ref[...], hist_s,
                    preferred_element_type=jnp.float32)
    carsum = jnp.sum(oh3f * carry[:, None, :], axis=-1)
    rank_ref[0] = (subrank + carsum).astype(jnp.int32)
    hist_ref[0, 0, :] = jnp.sum(hist_s, axis=0).astype(jnp.int32)


_p1 = pl.pallas_call(
    _p1_body,
    grid=(NBLK,),
    in_specs=[
        pl.BlockSpec((BLK, IN_F), lambda i: (i, 0)),
        pl.BlockSpec((IN_F, E), lambda i: (0, 0)),
        pl.BlockSpec((1, E), lambda i: (0, 0)),
        pl.BlockSpec((NSB, SB, SB), lambda i: (0, 0, 0)),
        pl.BlockSpec((NSB, NSB), lambda i: (0, 0)),
    ],
    out_specs=[
        pl.BlockSpec((BLK, E), lambda i: (i, 0)),
        pl.BlockSpec((1, NSB, SB), lambda i: (i, 0, 0)),
        pl.BlockSpec((1, NSB, SB), lambda i: (i, 0, 0)),
        pl.BlockSpec((1, 1, E), lambda i: (i, 0, 0)),
        pl.BlockSpec((BLK, IN_F // 2), lambda i: (i, 0)),
    ],
    out_shape=[
        jax.ShapeDtypeStruct((T, E), jnp.float32),
        jax.ShapeDtypeStruct((NBLK, NSB, SB), jnp.int32),
        jax.ShapeDtypeStruct((NBLK, NSB, SB), jnp.int32),
        jax.ShapeDtypeStruct((NBLK, 1, E), jnp.int32),
        jax.ShapeDtypeStruct((T, IN_F // 2), jnp.float32),
    ],
)


def _p2_body(hist_ref, eid_ref, rank_ref, pos_ref, te_ref):
    b = pl.program_id(0)
    h = hist_ref[...].reshape(NBLK, E).astype(jnp.float32)
    counts = jnp.sum(h, axis=0, keepdims=True)
    padded = jnp.ceil(counts / TILE_M) * TILE_M
    rr = lax.broadcasted_iota(jnp.int32, (E, E), 0)
    cc = lax.broadcasted_iota(jnp.int32, (E, E), 1)
    triu = (rr <= cc).astype(jnp.float32)
    pcum = jnp.dot(padded, triu, preferred_element_type=jnp.float32)
    poff = pcum - padded
    mask = (lax.broadcasted_iota(jnp.int32, (1, NBLK), 1) < b).astype(jnp.float32)
    carry = jnp.dot(mask, h, preferred_element_type=jnp.float32)
    db = poff + carry
    eidl = eid_ref[0]
    oh3 = (eidl[:, :, None] == lax.broadcasted_iota(jnp.int32, (1, 1, E), 2))
    dbb = db.reshape(1, 1, E)
    pos = jnp.sum(oh3.astype(jnp.float32) * dbb, axis=-1).astype(jnp.int32)
    pos_ref[0] = pos + rank_ref[0]
    @pl.when(b == 0)
    def _():
        f = (lax.broadcasted_iota(jnp.int32, (8, 128), 0) * 128
             + lax.broadcasted_iota(jnp.int32, (8, 128), 1))
        p = (f * TILE_M).astype(jnp.float32)
        acc = jnp.zeros((8, 128), jnp.float32)
        for e in range(E):
            pe = lax.slice(poff, (0, e), (1, e + 1))
            acc = acc + (p >= pe).astype(jnp.float32)
        te_ref[...] = (acc - 1.0).astype(jnp.int32)


_p2 = pl.pallas_call(
    _p2_body,
    grid=(NBLK,),
    in_specs=[
        pl.BlockSpec((NBLK, 1, E), lambda i: (0, 0, 0)),
        pl.BlockSpec((1, NSB, SB), lambda i: (i, 0, 0)),
        pl.BlockSpec((1, NSB, SB), lambda i: (i, 0, 0)),
    ],
    out_specs=[
        pl.BlockSpec((1, NSB, SB), lambda i: (i, 0, 0)),
        pl.BlockSpec((8, 128), lambda i: (0, 0)),
    ],
    out_shape=[
        jax.ShapeDtypeStruct((NBLK, NSB, SB), jnp.int32),
        jax.ShapeDtypeStruct((8, 128), jnp.int32),
    ],
)


def _p3_body(x_hbm, pos_hbm, xs_hbm, rows_v, pos_v,
             l0, l1, q0, q1, s0, s1):
    w = lax.axis_index("s") * 2 + lax.axis_index("c")
    base = w * TPW
    lsem, qsem, ssem = [l0, l1], [q0, q1], [s0, s1]
    nch = TPW // P3C

    def load(c):
        buf = c % 2
        lx = pltpu.async_copy(x_hbm.at[pl.ds(base + c * P3C, P3C)],
                              rows_v.at[buf], lsem[buf])
        lp = pltpu.async_copy(pos_hbm.at[pl.ds(base + c * P3C, P3C)],
                              pos_v.at[buf], qsem[buf])
        return lx, lp

    loads = {0: load(0)}
    scats = {}
    for c in range(nch):
        buf = c % 2
        lx, lp = loads.pop(c)
        lx.wait()
        lp.wait()
        if c + 1 < nch:
            if c - 1 >= 0:
                scats.pop(c - 1).wait()
            loads[c + 1] = load(c + 1)
        scats[c] = pltpu.async_copy(rows_v.at[buf],
                                    xs_hbm.at[pos_v.at[buf]], ssem[buf])
    for c in sorted(scats):
        scats.pop(c).wait()


@functools.cache
def _sc_kernels():
    mesh = plsc.VectorSubcoreMesh(core_axis_name="c", subcore_axis_name="s")
    p3 = pl.kernel(
        _p3_body,
        out_type=jax.ShapeDtypeStruct((PADDED, IN_F // 2), jnp.float32),
        mesh=mesh,
        scratch_types=(pltpu.VMEM((2, P3C, IN_F // 2), jnp.float32),
                       pltpu.VMEM((2, P3C), jnp.int32),
                       pltpu.SemaphoreType.DMA, pltpu.SemaphoreType.DMA,
                       pltpu.SemaphoreType.DMA, pltpu.SemaphoreType.DMA,
                       pltpu.SemaphoreType.DMA, pltpu.SemaphoreType.DMA),
    )
    p5 = pl.kernel(
        _p5_body,
        out_type=jax.ShapeDtypeStruct((T, OUT_F), jnp.float32),
        mesh=mesh,
        scratch_types=(pltpu.VMEM((2, P5C, OUT_F), jnp.float32),
                       pltpu.VMEM((2, P5C), jnp.int32),
                       pltpu.SemaphoreType.DMA, pltpu.SemaphoreType.DMA,
                       pltpu.SemaphoreType.DMA, pltpu.SemaphoreType.DMA),
    )
    return p3, p5


def _p4_body(tid_ref, xs_ref, w1_ref, b1_ref, w2_ref, b2_ref, ys_ref):
    xw = lax.bitcast_convert_type(xs_ref[...], jnp.int32)
    xe = lax.bitcast_convert_type(
        xw & jnp.int32(0xFFFF0000 - 0x100000000), jnp.float32)
    xo = lax.bitcast_convert_type(xw << 16, jnp.float32)
    w1 = w1_ref[0]
    h = jnp.dot(xe.astype(jnp.bfloat16), w1[:IN_F // 2],
                preferred_element_type=jnp.float32)
    h = h + jnp.dot(xo.astype(jnp.bfloat16), w1[IN_F // 2:],
                    preferred_element_type=jnp.float32)
    h = jnp.maximum(h + b1_ref[0], 0.0)
    y = jnp.dot(h.astype(jnp.bfloat16), w2_ref[0],
                preferred_element_type=jnp.float32)
    ys_ref[...] = y + b2_ref[0]


def _tid_map(i, tid):
    return (tid[i // 128, i % 128], 0, 0)


_p4 = pl.pallas_call(
    _p4_body,
    grid_spec=pltpu.PrefetchScalarGridSpec(
        num_scalar_prefetch=1,
        grid=(NTILES,),
        in_specs=[
            pl.BlockSpec((TILE_M, IN_F // 2), lambda i, tid: (i, 0)),
            pl.BlockSpec((1, IN_F, 2 * HID), _tid_map),
            pl.BlockSpec((1, 1, 2 * HID), _tid_map),
            pl.BlockSpec((1, 2 * HID, OUT_F), _tid_map),
            pl.BlockSpec((1, 1, OUT_F), _tid_map),
        ],
        out_specs=pl.BlockSpec((TILE_M, OUT_F), lambda i, tid: (i, 0)),
    ),
    out_shape=jax.ShapeDtypeStruct((PADDED, OUT_F), jnp.float32),
)


def _p5_body(ys_hbm, pos_hbm, out_hbm, yr_v, pos_v,
             g0, g1, t0, t1):
    w = lax.axis_index("s") * 2 + lax.axis_index("c")
    base = w * TPW
    gsem, tsem = [g0, g1], [t0, t1]
    nch = TPW // P5C

    pltpu.sync_copy(pos_hbm.at[pl.ds(base, P5C)], pos_v.at[0])
    gaths = {0: pltpu.async_copy(ys_hbm.at[pos_v.at[0]], yr_v.at[0], gsem[0])}
    stores = {}
    for c in range(nch):
        buf = c % 2
        gaths.pop(c).wait()
        stores[c] = pltpu.async_copy(
            yr_v.at[buf], out_hbm.at[pl.ds(base + c * P5C, P5C)], tsem[buf])
        if c + 1 < nch:
            ob = 1 - buf
            if c - 1 >= 0:
                stores.pop(c - 1).wait()
            pltpu.sync_copy(pos_hbm.at[pl.ds(base + (c + 1) * P5C, P5C)],
                            pos_v.at[ob])
            gaths[c + 1] = pltpu.async_copy(ys_hbm.at[pos_v.at[ob]],
                                            yr_v.at[ob], gsem[ob])
    for c in sorted(stores):
        stores.pop(c).wait()


def kernel(x, gate_w, gate_b, eW1, eb1, eW2, eb2, sW1, sb1, sW2, sb2):
    Bq, Sq, Dd = x.shape
    xf = x.reshape(T, IN_F)
    ew, eid3, rank3, hist3, xp = _p1(
        xf, gate_w, gate_b.reshape(1, E), jnp.asarray(_TRIL3),
        jnp.asarray(_TRIL8))
    pos3, te = _p2(hist3, eid3, rank3)
    pos = pos3.reshape(T)
    p3, p5 = _sc_kernels()
    xs = p3(xp, pos)
    w1c = jnp.concatenate(
        [eW1, jnp.broadcast_to(sW1[0][None], (E, IN_F, HID))],
        axis=2).astype(jnp.bfloat16)
    b1c = jnp.concatenate(
        [eb1, jnp.broadcast_to(sb1[0][None], (E, HID))], axis=1)
    w2c = jnp.concatenate(
        [eW2, jnp.broadcast_to(sW2[0][None], (E, HID, OUT_F))],
        axis=1).astype(jnp.bfloat16)
    b2c = eb2 + sb2[0][None]
    ys = _p4(te, xs, w1c, b1c.reshape(E, 1, 2 * HID), w2c,
             b2c.reshape(E, 1, OUT_F))
    out = p5(ys, pos)
    return out.reshape(Bq, Sq, OUT_F), ew
